# baseline (speedup 1.0000x reference)
import jax
import jax.numpy as jnp
from jax import lax
from jax.experimental import pallas as pl
from jax.experimental.pallas import tpu as pltpu
from jax.experimental.pallas import tpu_sc as plsc

N = 10000
E = 320000
D = 128
C = 40

NC = 2
NS = 16
H = D // NC
K = 80
E_PER_TILE = E // NS
CHUNKS = E_PER_TILE // K
NBUF = 4
GROUPS = CHUNKS // 2
N_PAD = 10240
ROWS_PER_TILE = N_PAD // NS
ZROWS = 128


def _sc_segment_sum_body(xview_hbm, src_hbm, dst_hbm, out_sums, out_cnt,
                         idx_src, idx_dst, bufs, ones, zbuf, zcnt,
                         shared_sums, shared_cnt, gsems, ssems, csems):
    cid = lax.axis_index("c")
    sid = lax.axis_index("s")

    def zrow_body(i, _):
        for j in range(H // 16):
            zbuf[i, pl.ds(j * 16, 16)] = jnp.zeros((16,), jnp.float32)
        return 0
    lax.fori_loop(0, ZROWS, zrow_body, 0)

    def zcnt_body(i, _):
        zcnt[pl.ds(pl.multiple_of(i * 16, 16), 16)] = jnp.zeros((16,), jnp.float32)
        return 0
    lax.fori_loop(0, ROWS_PER_TILE // 16, zcnt_body, 0)

    for j in range(128 // 16):
        ones[pl.ds(j * 16, 16)] = jnp.ones((16,), jnp.float32)

    for k in range(ROWS_PER_TILE // ZROWS):
        pltpu.sync_copy(zbuf, shared_sums.at[pl.ds(sid * ROWS_PER_TILE + k * ZROWS, ZROWS)])
    pltpu.sync_copy(zcnt, shared_cnt.at[pl.ds(sid * ROWS_PER_TILE, ROWS_PER_TILE)])

    pltpu.sync_copy(src_hbm.at[sid], idx_src)
    pltpu.sync_copy(dst_hbm.at[sid], idx_dst)

    plsc.subcore_barrier()

    cidv = jnp.full((16,), cid, jnp.int32)

    def prep_row(j):
        for t in range(K // 16):
            v = idx_src[j, pl.ds(16 * t, 16)]
            idx_src[j, pl.ds(16 * t, 16)] = v + v + cidv

    def gather_start(j, b):
        prep_row(j)
        pltpu.async_copy(xview_hbm.at[idx_src.at[j]], bufs[b], gsems[b])

    def gather_wait(j, b):
        pltpu.make_async_copy(
            xview_hbm.at[idx_src.at[j]], bufs[b], gsems[b]).wait()

    def scatter_start(j, b):
        pltpu.async_copy(bufs[b], shared_sums.at[idx_dst.at[j]], ssems[b],
                         add=True)

    def scatter_wait(j, b):
        pltpu.make_async_copy(
            bufs[b], shared_sums.at[idx_dst.at[j]], ssems[b]).wait()

    def count_start(j, b):
        pltpu.async_copy(ones.at[pl.ds(0, K)], shared_cnt.at[idx_dst.at[j]],
                         csems[b], add=True)

    def count_wait(j, b):
        pltpu.make_async_copy(
            ones.at[pl.ds(0, K)], shared_cnt.at[idx_dst.at[j]], csems[b]).wait()

    for b in range(NBUF):
        gather_start(b, b)

    def group(g, base):
        for i in range(2):
            j = 2 * g + i
            b = base + i
            gather_wait(j, b)
            scatter_start(j, b)

            @pl.when(cid == i % 2)
            def _():
                count_start(j, b)

        for i in range(2):
            j = 2 * g + i
            b = base + i
            scatter_wait(j, b)

            @pl.when(cid == i % 2)
            def _():
                count_wait(j, b)

            @pl.when(j + NBUF < CHUNKS)
            def _():
                gather_start(j + NBUF, b)

    def pair_body(gg, _):
        group(2 * gg, 0)
        group(2 * gg + 1, 2)
        return 0
    lax.fori_loop(0, GROUPS // 2, pair_body, 0)
    if GROUPS % 2:
        for i in range(2):
            j = CHUNKS - 2 + i
            gather_wait(j, i)
            scatter_start(j, i)

            @pl.when(cid == i % 2)
            def _():
                count_start(j, i)
        for i in range(2):
            j = CHUNKS - 2 + i
            scatter_wait(j, i)

            @pl.when(cid == i % 2)
            def _():
                count_wait(j, i)

    plsc.subcore_barrier()

    for k in range(ROWS_PER_TILE // ZROWS):
        r0 = sid * ROWS_PER_TILE + k * ZROWS
        pltpu.sync_copy(shared_sums.at[pl.ds(r0, ZROWS)],
                        out_sums.at[cid, pl.ds(r0, ZROWS)])
    pltpu.sync_copy(shared_cnt.at[pl.ds(sid * ROWS_PER_TILE, ROWS_PER_TILE)],
                    out_cnt.at[cid, pl.ds(sid * ROWS_PER_TILE, ROWS_PER_TILE)])


@jax.jit
def _sc_segment_sum(xview, src3d, dst3d):
    mesh = plsc.VectorSubcoreMesh(core_axis_name="c", subcore_axis_name="s")
    f = pl.kernel(
        _sc_segment_sum_body,
        out_type=[
            jax.ShapeDtypeStruct((NC, N_PAD, H), jnp.float32),
            jax.ShapeDtypeStruct((NC, N_PAD), jnp.float32),
        ],
        mesh=mesh,
        compiler_params=pltpu.CompilerParams(use_tc_tiling_on_sc=False),
        scratch_types=[
            pltpu.VMEM((CHUNKS, K), jnp.int32),
            pltpu.VMEM((CHUNKS, K), jnp.int32),
            [pltpu.VMEM((K, H), jnp.float32) for _ in range(NBUF)],
            pltpu.VMEM((128,), jnp.float32),
            pltpu.VMEM((ZROWS, H), jnp.float32),
            pltpu.VMEM((ROWS_PER_TILE,), jnp.float32),
            pltpu.VMEM_SHARED((N_PAD, H), jnp.float32),
            pltpu.VMEM_SHARED((N_PAD,), jnp.float32),
            [pltpu.SemaphoreType.DMA for _ in range(NBUF)],
            [pltpu.SemaphoreType.DMA for _ in range(NBUF)],
            [pltpu.SemaphoreType.DMA for _ in range(NBUF)],
        ],
    )
    return f(xview, src3d, dst3d)


def _dense_body(x_ref, pl_ref, pr_ref, cnt_ref, wes_ref, went_ref, wenb_ref,
                be_ref, wc_ref, bc_ref, out_ref):
    cnt = cnt_ref[:, 0] + cnt_ref[:, 1]
    inv = 1.0 / jnp.maximum(cnt, 1.0)
    agg = (pl_ref[0] * inv[:, None]) @ went_ref[...] \
        + (pr_ref[0] * inv[:, None]) @ wenb_ref[...]
    h = x_ref[...] @ wes_ref[...] + agg + be_ref[...]
    h = jax.nn.gelu(h)
    logits = h @ wc_ref[...] + bc_ref[...]
    out_ref[...] = jax.nn.softmax(jax.nn.gelu(logits), axis=-1)


def _dense(x, psums, cnt, wes, went, wenb, be, wc, bc):
    B = 5000
    grid = (N // B,)
    return pl.pallas_call(
        _dense_body,
        grid=grid,
        in_specs=[
            pl.BlockSpec((B, D), lambda i: (i, 0)),
            pl.BlockSpec((1, B, H), lambda i: (0, i, 0)),
            pl.BlockSpec((1, B, H), lambda i: (1, i, 0)),
            pl.BlockSpec((B, NC), lambda i: (i, 0)),
            pl.BlockSpec((D, D), lambda i: (0, 0)),
            pl.BlockSpec((H, D), lambda i: (0, 0)),
            pl.BlockSpec((H, D), lambda i: (0, 0)),
            pl.BlockSpec((1, D), lambda i: (0, 0)),
            pl.BlockSpec((D, C), lambda i: (0, 0)),
            pl.BlockSpec((1, C), lambda i: (0, 0)),
        ],
        out_specs=pl.BlockSpec((B, C), lambda i: (i, 0)),
        out_shape=jax.ShapeDtypeStruct((N, C), jnp.float32),
    )(x, psums, psums, cnt, wes, went, wenb, be, wc, bc)


def kernel(x, edge_index, W_env_self, W_env_nbr, b_env, W_act_self,
           W_act_nbr, b_act, W_cls, b_cls):
    xview = x.reshape(2 * N, H)
    src3d = edge_index[0].reshape(NS, CHUNKS, K)
    dst3d = edge_index[1].reshape(NS, CHUNKS, K)
    psums, pcnt = _sc_segment_sum(xview, src3d, dst3d)

    be = b_env.reshape(1, D)
    bc = b_cls.reshape(1, C)
    cnt = pcnt[:, :N].T

    return _dense(x, psums, cnt, W_env_self,
                  W_env_nbr[:H], W_env_nbr[H:], be, W_cls, bc)


# speedup vs baseline: 1.1291x; 1.1291x over previous
import jax
import jax.numpy as jnp
from jax import lax
from jax.experimental import pallas as pl
from jax.experimental.pallas import tpu as pltpu
from jax.experimental.pallas import tpu_sc as plsc

N = 10000
E = 320000
D = 128
C = 40

NC = 2
NS = 16
H = D // NC
K = 128
NBLK = E // K
BPT = NBLK // NS
TAIL0 = BPT * NS
CHUNKS = BPT
NBUF = 4
GROUPS = CHUNKS // 2
N_PAD = 10240
ROWS_PER_TILE = N_PAD // NS
ZROWS = 128


def _sc_segment_sum_body(xview_hbm, eidx_hbm, out_sums, out_cnt,
                         idx_e, bufs, ones, zbuf, zcnt,
                         shared_sums, shared_cnt, gsems, ssems, csems):
    cid = lax.axis_index("c")
    sid = lax.axis_index("s")

    def zrow_body(i, _):
        for j in range(H // 16):
            zbuf[i, pl.ds(j * 16, 16)] = jnp.zeros((16,), jnp.float32)
        return 0
    lax.fori_loop(0, ZROWS, zrow_body, 0)

    def zcnt_body(i, _):
        zcnt[pl.ds(pl.multiple_of(i * 16, 16), 16)] = jnp.zeros((16,), jnp.float32)
        return 0
    lax.fori_loop(0, ROWS_PER_TILE // 16, zcnt_body, 0)

    for j in range(128 // 16):
        ones[pl.ds(j * 16, 16)] = jnp.ones((16,), jnp.float32)

    for k in range(ROWS_PER_TILE // ZROWS):
        pltpu.sync_copy(zbuf, shared_sums.at[pl.ds(sid * ROWS_PER_TILE + k * ZROWS, ZROWS)])
    pltpu.sync_copy(zcnt, shared_cnt.at[pl.ds(sid * ROWS_PER_TILE, ROWS_PER_TILE)])

    pltpu.sync_copy(eidx_hbm.at[pl.ds(sid * BPT, BPT)], idx_e.at[pl.ds(0, BPT)])

    @pl.when(sid < 4)
    def _():
        pltpu.sync_copy(eidx_hbm.at[pl.ds(TAIL0 + sid, 1)], idx_e.at[pl.ds(BPT, 1)])

    plsc.subcore_barrier()

    cidv = jnp.full((16,), cid, jnp.int32)

    def prep_row(j):
        for t in range(K // 16):
            v = idx_e[j, 0, pl.ds(16 * t, 16)]
            idx_e[j, 0, pl.ds(16 * t, 16)] = v + v + cidv

    def gather_start(j, b):
        prep_row(j)
        pltpu.async_copy(xview_hbm.at[idx_e.at[j, 0]], bufs[b], gsems[b])

    def gather_wait(j, b):
        pltpu.make_async_copy(
            xview_hbm.at[idx_e.at[j, 0]], bufs[b], gsems[b]).wait()

    def scatter_start(j, b):
        pltpu.async_copy(bufs[b], shared_sums.at[idx_e.at[j, 1]], ssems[b],
                         add=True)

    def scatter_wait(j, b):
        pltpu.make_async_copy(
            bufs[b], shared_sums.at[idx_e.at[j, 1]], ssems[b]).wait()

    def count_start(j, b):
        pltpu.async_copy(ones, shared_cnt.at[idx_e.at[j, 1]],
                         csems[b], add=True)

    def count_wait(j, b):
        pltpu.make_async_copy(
            ones, shared_cnt.at[idx_e.at[j, 1]], csems[b]).wait()

    chunks_t = lax.select(sid < 4, jnp.int32(BPT + 1), jnp.int32(BPT))

    for b in range(NBUF):
        gather_start(b, b)

    def group(g, base):
        for i in range(2):
            j = 2 * g + i
            b = base + i
            gather_wait(j, b)
            scatter_start(j, b)

            @pl.when(cid == i % 2)
            def _():
                count_start(j, b)

        for i in range(2):
            j = 2 * g + i
            b = base + i
            scatter_wait(j, b)

            @pl.when(cid == i % 2)
            def _():
                count_wait(j, b)

            @pl.when(j + NBUF < chunks_t)
            def _():
                gather_start(j + NBUF, b)

    def pair_body(gg, _):
        group(2 * gg, 0)
        group(2 * gg + 1, 2)
        return 0
    lax.fori_loop(0, GROUPS // 2, pair_body, 0)

    @pl.when(sid < 4)
    def _():
        gather_wait(BPT, 0)
        scatter_start(BPT, 0)

        @pl.when(cid == 0)
        def _():
            count_start(BPT, 0)

        scatter_wait(BPT, 0)

        @pl.when(cid == 0)
        def _():
            count_wait(BPT, 0)

    plsc.subcore_barrier()

    for k in range(ROWS_PER_TILE // ZROWS):
        r0 = sid * ROWS_PER_TILE + k * ZROWS
        pltpu.sync_copy(shared_sums.at[pl.ds(r0, ZROWS)],
                        out_sums.at[cid, pl.ds(r0, ZROWS)])
    pltpu.sync_copy(shared_cnt.at[pl.ds(sid * ROWS_PER_TILE, ROWS_PER_TILE)],
                    out_cnt.at[cid, pl.ds(sid * ROWS_PER_TILE, ROWS_PER_TILE)])


@jax.jit
def _sc_segment_sum(xview, eidx):
    mesh = plsc.VectorSubcoreMesh(core_axis_name="c", subcore_axis_name="s")
    f = pl.kernel(
        _sc_segment_sum_body,
        out_type=[
            jax.ShapeDtypeStruct((NC, N_PAD, H), jnp.float32),
            jax.ShapeDtypeStruct((NC, N_PAD), jnp.float32),
        ],
        mesh=mesh,
        compiler_params=pltpu.CompilerParams(use_tc_tiling_on_sc=False),
        scratch_types=[
            pltpu.VMEM((BPT + 1, 2, K), jnp.int32),
            [pltpu.VMEM((K, H), jnp.float32) for _ in range(NBUF)],
            pltpu.VMEM((128,), jnp.float32),
            pltpu.VMEM((ZROWS, H), jnp.float32),
            pltpu.VMEM((ROWS_PER_TILE,), jnp.float32),
            pltpu.VMEM_SHARED((N_PAD, H), jnp.float32),
            pltpu.VMEM_SHARED((N_PAD,), jnp.float32),
            [pltpu.SemaphoreType.DMA for _ in range(NBUF)],
            [pltpu.SemaphoreType.DMA for _ in range(NBUF)],
            [pltpu.SemaphoreType.DMA for _ in range(NBUF)],
        ],
    )
    return f(xview, eidx)


def _dense_body(x_ref, pl_ref, pr_ref, cnt_ref, wes_ref, went_ref, wenb_ref,
                be_ref, wc_ref, bc_ref, out_ref):
    cnt = cnt_ref[:, 0] + cnt_ref[:, 1]
    inv = 1.0 / jnp.maximum(cnt, 1.0)
    agg = (pl_ref[0] * inv[:, None]) @ went_ref[...] \
        + (pr_ref[0] * inv[:, None]) @ wenb_ref[...]
    h = x_ref[...] @ wes_ref[...] + agg + be_ref[...]
    h = jax.nn.gelu(h)
    logits = h @ wc_ref[...] + bc_ref[...]
    out_ref[...] = jax.nn.softmax(jax.nn.gelu(logits), axis=-1)


def _dense(x, psums, cnt, wes, went, wenb, be, wc, bc):
    B = 5000
    grid = (N // B,)
    return pl.pallas_call(
        _dense_body,
        grid=grid,
        in_specs=[
            pl.BlockSpec((B, D), lambda i: (i, 0)),
            pl.BlockSpec((1, B, H), lambda i: (0, i, 0)),
            pl.BlockSpec((1, B, H), lambda i: (1, i, 0)),
            pl.BlockSpec((B, NC), lambda i: (i, 0)),
            pl.BlockSpec((D, D), lambda i: (0, 0)),
            pl.BlockSpec((H, D), lambda i: (0, 0)),
            pl.BlockSpec((H, D), lambda i: (0, 0)),
            pl.BlockSpec((1, D), lambda i: (0, 0)),
            pl.BlockSpec((D, C), lambda i: (0, 0)),
            pl.BlockSpec((1, C), lambda i: (0, 0)),
        ],
        out_specs=pl.BlockSpec((B, C), lambda i: (i, 0)),
        out_shape=jax.ShapeDtypeStruct((N, C), jnp.float32),
    )(x, psums, psums, cnt, wes, went, wenb, be, wc, bc)


def kernel(x, edge_index, W_env_self, W_env_nbr, b_env, W_act_self,
           W_act_nbr, b_act, W_cls, b_cls):
    xview = x.reshape(2 * N, H)
    eidx = edge_index.reshape(2, NBLK, K).transpose(1, 0, 2)
    psums, pcnt = _sc_segment_sum(xview, eidx)

    be = b_env.reshape(1, D)
    bc = b_cls.reshape(1, C)
    cnt = pcnt[:, :N].T

    return _dense(x, psums, cnt, W_env_self,
                  W_env_nbr[:H], W_env_nbr[H:], be, W_cls, bc)


# speedup vs baseline: 1.2069x; 1.0689x over previous
import jax
import jax.numpy as jnp
from jax import lax
from jax.experimental import pallas as pl
from jax.experimental.pallas import tpu as pltpu
from jax.experimental.pallas import tpu_sc as plsc

N = 10000
E = 320000
D = 128
C = 40

NC = 2
NS = 16
H = D // NC
K = 128
NBLK = E // K
BPT = NBLK // NS
TAIL0 = BPT * NS
CHUNKS = BPT
NBUF = 4
GROUPS = CHUNKS // 2
N_PAD = 10240
ROWS_PER_TILE = N_PAD // NS
ZROWS = 128


def _sc_segment_sum_body(xview_hbm, eidx_hbm, out_sums, out_cnt,
                         idx_e, bufs, ones, zbuf, zcnt,
                         shared_sums, shared_cnt, gsems, ssems, csems):
    cid = lax.axis_index("c")
    sid = lax.axis_index("s")

    def zrow_body(i, _):
        for j in range(H // 16):
            zbuf[i, pl.ds(j * 16, 16)] = jnp.zeros((16,), jnp.float32)
        return 0
    lax.fori_loop(0, ZROWS, zrow_body, 0)

    def zcnt_body(i, _):
        zcnt[pl.ds(pl.multiple_of(i * 16, 16), 16)] = jnp.zeros((16,), jnp.float32)
        return 0
    lax.fori_loop(0, ROWS_PER_TILE // 16, zcnt_body, 0)

    for j in range(128 // 16):
        ones[pl.ds(j * 16, 16)] = jnp.ones((16,), jnp.float32)

    for k in range(ROWS_PER_TILE // ZROWS):
        pltpu.sync_copy(zbuf, shared_sums.at[pl.ds(sid * ROWS_PER_TILE + k * ZROWS, ZROWS)])
    pltpu.sync_copy(zcnt, shared_cnt.at[pl.ds(sid * ROWS_PER_TILE, ROWS_PER_TILE)])

    pltpu.sync_copy(eidx_hbm.at[pl.ds(sid * BPT, BPT)], idx_e.at[pl.ds(0, BPT)])

    @pl.when(sid < 4)
    def _():
        pltpu.sync_copy(eidx_hbm.at[pl.ds(TAIL0 + sid, 1)], idx_e.at[pl.ds(BPT, 1)])

    plsc.subcore_barrier()

    cidv = jnp.full((16,), cid, jnp.int32)

    def prep_row(j):
        for t in range(K // 16):
            v = idx_e[j, 0, pl.ds(16 * t, 16)]
            idx_e[j, 0, pl.ds(16 * t, 16)] = v + v + cidv

    def gather_start(j, b):
        prep_row(j)
        pltpu.async_copy(xview_hbm.at[idx_e.at[j, 0]], bufs[b], gsems[b])

    def gather_wait(j, b):
        pltpu.make_async_copy(
            xview_hbm.at[idx_e.at[j, 0]], bufs[b], gsems[b]).wait()

    def scatter_start(j, b):
        pltpu.async_copy(bufs[b], shared_sums.at[idx_e.at[j, 1]], ssems[b],
                         add=True)

    def scatter_wait(j, b):
        pltpu.make_async_copy(
            bufs[b], shared_sums.at[idx_e.at[j, 1]], ssems[b]).wait()

    def count_start(j, b):
        pltpu.async_copy(ones, shared_cnt.at[idx_e.at[j, 1]],
                         csems[b], add=True)

    def count_wait(j, b):
        pltpu.make_async_copy(
            ones, shared_cnt.at[idx_e.at[j, 1]], csems[b]).wait()

    chunks_t = lax.select(sid < 4, jnp.int32(BPT + 1), jnp.int32(BPT))

    for b in range(NBUF):
        gather_start(b, b)

    def group(g, base):
        for i in range(2):
            j = 2 * g + i
            b = base + i
            gather_wait(j, b)
            scatter_start(j, b)

            @pl.when(cid == i % 2)
            def _():
                count_start(j, b)

        for i in range(2):
            j = 2 * g + i
            b = base + i
            scatter_wait(j, b)

            @pl.when(cid == i % 2)
            def _():
                count_wait(j, b)

            @pl.when(j + NBUF < chunks_t)
            def _():
                gather_start(j + NBUF, b)

    def pair_body(gg, _):
        group(2 * gg, 0)
        group(2 * gg + 1, 2)
        return 0
    lax.fori_loop(0, GROUPS // 2, pair_body, 0)

    @pl.when(sid < 4)
    def _():
        gather_wait(BPT, 0)
        scatter_start(BPT, 0)

        @pl.when(cid == 0)
        def _():
            count_start(BPT, 0)

        scatter_wait(BPT, 0)

        @pl.when(cid == 0)
        def _():
            count_wait(BPT, 0)

    plsc.subcore_barrier()

    for k in range(ROWS_PER_TILE // ZROWS):
        r0 = sid * ROWS_PER_TILE + k * ZROWS
        pltpu.sync_copy(shared_sums.at[pl.ds(r0, ZROWS)],
                        out_sums.at[cid, pl.ds(r0, ZROWS)])
    pltpu.sync_copy(shared_cnt.at[pl.ds(sid * ROWS_PER_TILE, ROWS_PER_TILE)],
                    out_cnt.at[cid, pl.ds(sid * ROWS_PER_TILE, ROWS_PER_TILE)])


@jax.jit
def _sc_segment_sum(xview, eidx):
    mesh = plsc.VectorSubcoreMesh(core_axis_name="c", subcore_axis_name="s")
    f = pl.kernel(
        _sc_segment_sum_body,
        out_type=[
            jax.ShapeDtypeStruct((NC, N_PAD, H), jnp.float32),
            jax.ShapeDtypeStruct((NC, N_PAD), jnp.float32),
        ],
        mesh=mesh,
        compiler_params=pltpu.CompilerParams(use_tc_tiling_on_sc=False),
        scratch_types=[
            pltpu.VMEM((BPT + 1, 2, K), jnp.int32),
            [pltpu.VMEM((K, H), jnp.float32) for _ in range(NBUF)],
            pltpu.VMEM((128,), jnp.float32),
            pltpu.VMEM((ZROWS, H), jnp.float32),
            pltpu.VMEM((ROWS_PER_TILE,), jnp.float32),
            pltpu.VMEM_SHARED((N_PAD, H), jnp.float32),
            pltpu.VMEM_SHARED((N_PAD,), jnp.float32),
            [pltpu.SemaphoreType.DMA for _ in range(NBUF)],
            [pltpu.SemaphoreType.DMA for _ in range(NBUF)],
            [pltpu.SemaphoreType.DMA for _ in range(NBUF)],
        ],
    )
    return f(xview, eidx)


def _dense_body(x_ref, pl_ref, pr_ref, cnt_ref, wes_ref, went_ref, wenb_ref,
                be_ref, wc_ref, bc_ref, out_ref):
    cnt = cnt_ref[0, :] + cnt_ref[1, :]
    inv = 1.0 / jnp.maximum(cnt, 1.0)
    agg = (pl_ref[0] * inv[:, None]) @ went_ref[...] \
        + (pr_ref[0] * inv[:, None]) @ wenb_ref[...]
    h = x_ref[...] @ wes_ref[...] + agg + be_ref[...]
    h = jax.nn.gelu(h)
    logits = h @ wc_ref[...] + bc_ref[...]
    out_ref[...] = jax.nn.softmax(jax.nn.gelu(logits), axis=-1).T


def _dense(x, psums, pcnt, wes, went, wenb, be, wc, bc):
    B = 5120
    grid = (2,)
    return pl.pallas_call(
        _dense_body,
        grid=grid,
        in_specs=[
            pl.BlockSpec((B, D), lambda i: (i, 0)),
            pl.BlockSpec((1, B, H), lambda i: (0, i, 0)),
            pl.BlockSpec((1, B, H), lambda i: (1, i, 0)),
            pl.BlockSpec((NC, B), lambda i: (0, i)),
            pl.BlockSpec((D, D), lambda i: (0, 0)),
            pl.BlockSpec((H, D), lambda i: (0, 0)),
            pl.BlockSpec((H, D), lambda i: (0, 0)),
            pl.BlockSpec((1, D), lambda i: (0, 0)),
            pl.BlockSpec((D, C), lambda i: (0, 0)),
            pl.BlockSpec((1, C), lambda i: (0, 0)),
        ],
        out_specs=pl.BlockSpec((C, B), lambda i: (0, i)),
        out_shape=jax.ShapeDtypeStruct((C, N), jnp.float32),
    )(x, psums, psums, pcnt, wes, went, wenb, be, wc, bc)


def kernel(x, edge_index, W_env_self, W_env_nbr, b_env, W_act_self,
           W_act_nbr, b_act, W_cls, b_cls):
    xview = x.reshape(2 * N, H)
    eidx = edge_index.reshape(2, NBLK, K).transpose(1, 0, 2)
    psums, pcnt = _sc_segment_sum(xview, eidx)

    be = b_env.reshape(1, D)
    bc = b_cls.reshape(1, C)

    return _dense(x, psums, pcnt, W_env_self,
                  W_env_nbr[:H], W_env_nbr[H:], be, W_cls, bc).T
